# baseline (speedup 1.0000x reference)
import functools

import jax
import jax.numpy as jnp
from jax import lax
from jax.experimental import pallas as pl
from jax.experimental.pallas import tpu as pltpu
from jax.experimental.pallas import tpu_sc as plsc

_LANE = 128
_NW = 32
_SCL = 16


def _pick_rb(n, cap=512):
    best = 8
    for rb in range(8, cap + 1, 8):
        if n % rb == 0:
            best = rb
    return best


def _dotb(a, b):
    return jnp.dot(a.astype(jnp.bfloat16), b.astype(jnp.bfloat16),
                   preferred_element_type=jnp.float32)


def _proj_kernel(x_ref, w_ref, b_ref, o_ref):
    o_ref[...] = _dotb(x_ref[...], w_ref[...]) + b_ref[...]


def _proj(x, w, b, rb):
    n, f = x.shape
    c = w.shape[1]
    return pl.pallas_call(
        _proj_kernel,
        grid=(n // rb,),
        in_specs=[
            pl.BlockSpec((rb, f), lambda i: (i, 0)),
            pl.BlockSpec((f, c), lambda i: (0, 0)),
            pl.BlockSpec((1, c), lambda i: (0, 0)),
        ],
        out_specs=pl.BlockSpec((rb, c), lambda i: (i, 0)),
        out_shape=jax.ShapeDtypeStruct((n, c), jnp.float32),
    )(x, w, b.reshape(1, c))


def _knn_kernel(sr_ref, scT_ref, o_ref):
    sr = sr_ref[...]
    scT = scT_ref[...]
    s_c = scT[0:4, :]
    s_r = sr[:, 0:4]
    s2c = jnp.sum(s_c * s_c, axis=0, keepdims=True)
    s2r = jnp.sum(s_r * s_r, axis=1, keepdims=True)
    dot = jax.lax.dot_general(
        s_r.astype(jnp.bfloat16), s_c.astype(jnp.bfloat16),
        (((1,), (0,)), ((), ())),
        preferred_element_type=jnp.float32)
    d2 = s2r - 2.0 * dot + s2c
    iota = jax.lax.broadcasted_iota(jnp.int32, d2.shape, 1)
    big = jnp.int32(2**30)
    inf = jnp.float32(jnp.inf)
    idxs = []
    for k in range(3):
        m = jnp.min(d2, axis=1, keepdims=True)
        sel = jnp.where(d2 == m, iota, big)
        idx = jnp.min(sel, axis=1, keepdims=True)
        if k < 2:
            d2 = jnp.where(iota == idx, inf, d2)
        idxs.append(idx)
    zero = jnp.zeros_like(idxs[0])
    o_ref[...] = jnp.concatenate(idxs + [zero] * 5, axis=1)


def _knn_idx3(sh_rows, scT, rb):
    n = sh_rows.shape[0]
    c = sh_rows.shape[1]
    np_ = scT.shape[1]
    return pl.pallas_call(
        _knn_kernel,
        grid=(n // rb,),
        in_specs=[
            pl.BlockSpec((rb, c), lambda i: (i, 0)),
            pl.BlockSpec((8, np_), lambda i: (0, 0)),
        ],
        out_specs=pl.BlockSpec((rb, 8), lambda i: (i, 0)),
        out_shape=jax.ShapeDtypeStruct((n, 8), jnp.int32),
    )(sh_rows, scT)


def _make_sc_agg(npad):
    chunk = npad // _NW
    groups = chunk // _SCL
    mesh = plsc.VectorSubcoreMesh(core_axis_name="c", subcore_axis_name="s")

    @functools.partial(
        pl.kernel, mesh=mesh,
        compiler_params=pltpu.CompilerParams(needs_layout_passes=False),
        out_type=jax.ShapeDtypeStruct((npad, 8), jnp.float32),
        scratch_types=[
            pltpu.VMEM((8 * npad,), jnp.float32),
            pltpu.VMEM((3 * chunk,), jnp.int32),
            pltpu.VMEM((chunk, 8), jnp.float32),
        ],
    )
    def sc_agg(table_hbm, idx_hbm, out_hbm, table_v, idx_v, out_v):
        wid = lax.axis_index("s") * 2 + lax.axis_index("c")
        base = wid * chunk
        pltpu.sync_copy(table_hbm, table_v)
        for k in range(3):
            pltpu.sync_copy(idx_hbm.at[pl.ds(k * npad + base, chunk)],
                            idx_v.at[pl.ds(k * chunk, chunk)])

        def body(g, carry):
            off = g * _SCL
            lanes = lax.iota(jnp.int32, _SCL)
            rows = lanes + off
            srd = [table_v[pl.ds(d * npad + base + off, _SCL)] for d in range(4)]
            mean_d = [None] * 3
            max_d = [None] * 3
            for k in range(3):
                jv = idx_v[pl.ds(k * chunk + off, _SCL)]
                picked = [plsc.load_gather(table_v, [jv + jnp.int32(d * npad)])
                          for d in range(7)]
                d2x = jnp.zeros((_SCL,), jnp.float32)
                for d in range(4):
                    df = srd[d] - picked[d]
                    d2x = d2x + df * df
                w = jnp.exp(-10.0 * d2x)
                for d in range(3):
                    msg = picked[4 + d] * w
                    if k == 0:
                        mean_d[d] = msg
                        max_d[d] = msg
                    else:
                        mean_d[d] = mean_d[d] + msg
                        max_d[d] = jnp.maximum(max_d[d], msg)
            zeros = jnp.zeros((_SCL,), jnp.float32)
            for d in range(3):
                plsc.store_scatter(out_v, [rows, jnp.full((_SCL,), d, jnp.int32)],
                                   mean_d[d] / 3.0)
                plsc.store_scatter(out_v, [rows, jnp.full((_SCL,), 3 + d, jnp.int32)],
                                   max_d[d])
            plsc.store_scatter(out_v, [rows, jnp.full((_SCL,), 6, jnp.int32)], zeros)
            plsc.store_scatter(out_v, [rows, jnp.full((_SCL,), 7, jnp.int32)], zeros)
            return carry

        lax.fori_loop(0, groups, body, jnp.int32(0))
        pltpu.sync_copy(out_v, out_hbm.at[pl.ds(base, chunk), :])

    return sc_agg


def _out_kernel(x_ref, agg_ref, w1_ref, w2_ref, b_ref, o_ref):
    acc = _dotb(agg_ref[...], w2_ref[...]) + b_ref[...]
    acc = _dotb(x_ref[...], w1_ref[...]) + acc
    o_ref[...] = jnp.maximum(acc, 0.0)


def _out_proj_kernel(x_ref, agg_ref, w1_ref, w2_ref, b_ref, wc_ref, bc_ref,
                     o_ref):
    acc = _dotb(agg_ref[...], w2_ref[...]) + b_ref[...]
    acc = _dotb(x_ref[...], w1_ref[...]) + acc
    h = jnp.maximum(acc, 0.0)
    o_ref[...] = _dotb(h, wc_ref[...]) + bc_ref[...]


def _out_proj_layer(x, agg, w1, w2e, b, wc, bc, rb):
    n, f = x.shape
    c = w1.shape[1]
    c2 = wc.shape[1]
    return pl.pallas_call(
        _out_proj_kernel,
        grid=(n // rb,),
        in_specs=[
            pl.BlockSpec((rb, f), lambda i: (i, 0)),
            pl.BlockSpec((rb, 8), lambda i: (i, 0)),
            pl.BlockSpec((f, c), lambda i: (0, 0)),
            pl.BlockSpec((8, c), lambda i: (0, 0)),
            pl.BlockSpec((1, c), lambda i: (0, 0)),
            pl.BlockSpec((c, c2), lambda i: (0, 0)),
            pl.BlockSpec((1, c2), lambda i: (0, 0)),
        ],
        out_specs=pl.BlockSpec((rb, c2), lambda i: (i, 0)),
        out_shape=jax.ShapeDtypeStruct((n, c2), jnp.float32),
    )(x, agg, w1, w2e, b.reshape(1, c), wc, bc.reshape(1, c2))


def _out_layer(x, agg, w1, w2e, b, rb):
    n, f = x.shape
    c = w1.shape[1]
    return pl.pallas_call(
        _out_kernel,
        grid=(n // rb,),
        in_specs=[
            pl.BlockSpec((rb, f), lambda i: (i, 0)),
            pl.BlockSpec((rb, 8), lambda i: (i, 0)),
            pl.BlockSpec((f, c), lambda i: (0, 0)),
            pl.BlockSpec((8, c), lambda i: (0, 0)),
            pl.BlockSpec((1, c), lambda i: (0, 0)),
        ],
        out_specs=pl.BlockSpec((rb, c), lambda i: (i, 0)),
        out_shape=jax.ShapeDtypeStruct((n, c), jnp.float32),
    )(x, agg, w1, w2e, b.reshape(1, c))


def _out2_kernel(y_ref, agg_ref, w2_ref, b_ref, o_ref):
    acc = y_ref[...] + (_dotb(agg_ref[...], w2_ref[...]) + b_ref[...])
    o_ref[...] = jnp.maximum(acc, 0.0)


def _out_layer2(y, agg, w2e, b, rb):
    n, c = y.shape
    return pl.pallas_call(
        _out2_kernel,
        grid=(n // rb,),
        in_specs=[
            pl.BlockSpec((rb, c), lambda i: (i, 0)),
            pl.BlockSpec((rb, 8), lambda i: (i, 0)),
            pl.BlockSpec((8, c), lambda i: (0, 0)),
            pl.BlockSpec((1, c), lambda i: (0, 0)),
        ],
        out_specs=pl.BlockSpec((rb, c), lambda i: (i, 0)),
        out_shape=jax.ShapeDtypeStruct((n, c), jnp.float32),
    )(y, agg, w2e, b.reshape(1, c))


def _make_scT(sh, n, npad):
    colsT = sh[:, 0:8].T if sh.shape[1] >= 8 else jnp.pad(sh, ((0, 0), (0, 8 - sh.shape[1]))).T
    pad = jnp.concatenate(
        [jnp.full((4, npad - n), 1e15, jnp.float32),
         jnp.zeros((4, npad - n), jnp.float32)], axis=0)
    return jnp.concatenate([colsT, pad], axis=1)


def _pad6to8(w2):
    return jnp.pad(w2, ((0, 2), (0, 0)))


def _knn_layer(sh, n, npad, rb_knn, sc_agg):
    scT = _make_scT(sh, n, npad)
    idx3 = _knn_idx3(sh, scT, rb_knn)
    idxT = jnp.concatenate(
        [idx3[:, 0:3].T, jnp.zeros((3, npad - n), jnp.int32)], axis=1)
    agg = sc_agg(scT.reshape(8 * npad), idxT.reshape(3 * npad))
    return agg[0:n]


def kernel(x, edge_index, Ws1, bs1, Wh1, bh1, Wo11, Wo21, bo21,
           Ws2, bs2, Wh2, bh2, Wo12, Wo22, bo22):
    del edge_index
    n = x.shape[0]
    npad = ((n + 511) // 512) * 512
    rb = _pick_rb(n, 400)
    rb_knn = _pick_rb(n, 1000)
    sc_agg = _make_sc_agg(npad)

    wsh1 = jnp.concatenate(
        [Ws1, Wh1, jnp.zeros((Ws1.shape[0], 1), jnp.float32)], axis=1)
    bsh1 = jnp.concatenate([bs1, bh1, jnp.zeros((1,), jnp.float32)])
    sh1 = _proj(x, wsh1, bsh1, _pick_rb(n, 1000))
    agg1 = _knn_layer(sh1, n, npad, rb_knn, sc_agg)

    hid = Wo11.shape[1]
    out_ch = Wo12.shape[1]
    w2cat = jnp.concatenate(
        [Ws2, Wh2, jnp.zeros((hid, 1), jnp.float32), Wo12], axis=1)
    b2cat = jnp.concatenate(
        [bs2, bh2, jnp.zeros((1 + out_ch,), jnp.float32)])
    shy2 = _out_proj_layer(x, agg1, Wo11, _pad6to8(Wo21), bo21,
                           w2cat, b2cat, rb)
    agg2 = _knn_layer(shy2, n, npad, rb_knn, sc_agg)
    y2a = shy2[:, 8:8 + out_ch]
    out = _out_layer2(y2a, agg2, _pad6to8(Wo22), bo22, _pick_rb(n, 1000))
    return out


# speedup vs baseline: 1.0449x; 1.0449x over previous
import functools

import jax
import jax.numpy as jnp
from jax import lax
from jax.experimental import pallas as pl
from jax.experimental.pallas import tpu as pltpu
from jax.experimental.pallas import tpu_sc as plsc

_LANE = 128
_NW = 32
_SCL = 16


def _pick_rb(n, cap=512):
    best = 8
    for rb in range(8, cap + 1, 8):
        if n % rb == 0:
            best = rb
    return best


def _dotb(a, b):
    return jnp.dot(a.astype(jnp.bfloat16), b.astype(jnp.bfloat16),
                   preferred_element_type=jnp.float32)


def _proj_kernel(x_ref, w_ref, b_ref, o_ref):
    o_ref[...] = _dotb(x_ref[...], w_ref[...]) + b_ref[...]


def _proj(x, w, b, rb):
    n, f = x.shape
    c = w.shape[1]
    return pl.pallas_call(
        _proj_kernel,
        grid=(n // rb,),
        in_specs=[
            pl.BlockSpec((rb, f), lambda i: (i, 0)),
            pl.BlockSpec((f, c), lambda i: (0, 0)),
            pl.BlockSpec((1, c), lambda i: (0, 0)),
        ],
        out_specs=pl.BlockSpec((rb, c), lambda i: (i, 0)),
        out_shape=jax.ShapeDtypeStruct((n, c), jnp.float32),
    )(x, w, b.reshape(1, c))


def _knn_kernel(sr_ref, scT_ref, o_ref):
    sr = sr_ref[...]
    scT = scT_ref[...]
    s_c = scT[0:4, :]
    s_r = sr[:, 0:4]
    s2c = jnp.sum(s_c * s_c, axis=0, keepdims=True)
    s2r = jnp.sum(s_r * s_r, axis=1, keepdims=True)
    dot = jax.lax.dot_general(
        s_r.astype(jnp.bfloat16), s_c.astype(jnp.bfloat16),
        (((1,), (0,)), ((), ())),
        preferred_element_type=jnp.float32)
    d2 = s2r - 2.0 * dot + s2c
    iota = jax.lax.broadcasted_iota(jnp.int32, d2.shape, 1)
    big = jnp.int32(2**30)
    inf = jnp.float32(jnp.inf)
    idxs = []
    for k in range(3):
        m = jnp.min(d2, axis=1, keepdims=True)
        sel = jnp.where(d2 == m, iota, big)
        idx = jnp.min(sel, axis=1, keepdims=True)
        if k < 2:
            d2 = jnp.where(iota == idx, inf, d2)
        idxs.append(idx)
    zero = jnp.zeros_like(idxs[0])
    o_ref[...] = jnp.concatenate(idxs + [zero] * 5, axis=1)


def _knn_idx3(sh_rows, scT, rb):
    n = sh_rows.shape[0]
    c = sh_rows.shape[1]
    np_ = scT.shape[1]
    return pl.pallas_call(
        _knn_kernel,
        grid=(n // rb,),
        in_specs=[
            pl.BlockSpec((rb, c), lambda i: (i, 0)),
            pl.BlockSpec((8, np_), lambda i: (0, 0)),
        ],
        out_specs=pl.BlockSpec((rb, 8), lambda i: (i, 0)),
        out_shape=jax.ShapeDtypeStruct((n, 8), jnp.int32),
    )(sh_rows, scT)


def _make_sc_agg(npad):
    chunk = npad // _NW
    groups = chunk // _SCL
    mesh = plsc.VectorSubcoreMesh(core_axis_name="c", subcore_axis_name="s")

    @functools.partial(
        pl.kernel, mesh=mesh,
        compiler_params=pltpu.CompilerParams(needs_layout_passes=False),
        out_type=jax.ShapeDtypeStruct((npad, 8), jnp.float32),
        scratch_types=[
            pltpu.VMEM((8 * npad,), jnp.float32),
            pltpu.VMEM((3 * chunk,), jnp.int32),
            pltpu.VMEM((chunk, 8), jnp.float32),
        ],
    )
    def sc_agg(table_hbm, idx_hbm, out_hbm, table_v, idx_v, out_v):
        wid = lax.axis_index("s") * 2 + lax.axis_index("c")
        base = wid * chunk
        pltpu.sync_copy(table_hbm, table_v)
        for k in range(3):
            pltpu.sync_copy(idx_hbm.at[pl.ds(k * npad + base, chunk)],
                            idx_v.at[pl.ds(k * chunk, chunk)])

        def body(g, carry):
            off = g * _SCL
            lanes = lax.iota(jnp.int32, _SCL)
            rows = lanes + off
            srd = [table_v[pl.ds(d * npad + base + off, _SCL)] for d in range(4)]
            mean_d = [None] * 3
            max_d = [None] * 3
            for k in range(3):
                jv = idx_v[pl.ds(k * chunk + off, _SCL)]
                picked = [plsc.load_gather(table_v, [jv + jnp.int32(d * npad)])
                          for d in range(7)]
                d2x = jnp.zeros((_SCL,), jnp.float32)
                for d in range(4):
                    df = srd[d] - picked[d]
                    d2x = d2x + df * df
                w = jnp.exp(-10.0 * d2x)
                for d in range(3):
                    msg = picked[4 + d] * w
                    if k == 0:
                        mean_d[d] = msg
                        max_d[d] = msg
                    else:
                        mean_d[d] = mean_d[d] + msg
                        max_d[d] = jnp.maximum(max_d[d], msg)
            zeros = jnp.zeros((_SCL,), jnp.float32)
            for d in range(3):
                plsc.store_scatter(out_v, [rows, jnp.full((_SCL,), d, jnp.int32)],
                                   mean_d[d] / 3.0)
                plsc.store_scatter(out_v, [rows, jnp.full((_SCL,), 3 + d, jnp.int32)],
                                   max_d[d])
            plsc.store_scatter(out_v, [rows, jnp.full((_SCL,), 6, jnp.int32)], zeros)
            plsc.store_scatter(out_v, [rows, jnp.full((_SCL,), 7, jnp.int32)], zeros)
            return carry

        lax.fori_loop(0, groups, body, jnp.int32(0))
        pltpu.sync_copy(out_v, out_hbm.at[pl.ds(base, chunk), :])

    return sc_agg


def _out_kernel(x_ref, agg_ref, w1_ref, w2_ref, b_ref, o_ref):
    acc = _dotb(agg_ref[...], w2_ref[...]) + b_ref[...]
    acc = _dotb(x_ref[...], w1_ref[...]) + acc
    o_ref[...] = jnp.maximum(acc, 0.0)


def _out_proj_kernel(x_ref, agg_ref, w1_ref, w2_ref, b_ref, wc_ref, bc_ref,
                     o_ref):
    acc = _dotb(agg_ref[...], w2_ref[...]) + b_ref[...]
    acc = _dotb(x_ref[...], w1_ref[...]) + acc
    h = jnp.maximum(acc, 0.0)
    o_ref[...] = _dotb(h, wc_ref[...]) + bc_ref[...]


def _out_proj_layer(x, agg, w1, w2e, b, wc, bc, rb):
    n, f = x.shape
    c = w1.shape[1]
    c2 = wc.shape[1]
    return pl.pallas_call(
        _out_proj_kernel,
        grid=(n // rb,),
        in_specs=[
            pl.BlockSpec((rb, f), lambda i: (i, 0)),
            pl.BlockSpec((rb, 8), lambda i: (i, 0)),
            pl.BlockSpec((f, c), lambda i: (0, 0)),
            pl.BlockSpec((8, c), lambda i: (0, 0)),
            pl.BlockSpec((1, c), lambda i: (0, 0)),
            pl.BlockSpec((c, c2), lambda i: (0, 0)),
            pl.BlockSpec((1, c2), lambda i: (0, 0)),
        ],
        out_specs=pl.BlockSpec((rb, c2), lambda i: (i, 0)),
        out_shape=jax.ShapeDtypeStruct((n, c2), jnp.float32),
    )(x, agg, w1, w2e, b.reshape(1, c), wc, bc.reshape(1, c2))


def _out_layer(x, agg, w1, w2e, b, rb):
    n, f = x.shape
    c = w1.shape[1]
    return pl.pallas_call(
        _out_kernel,
        grid=(n // rb,),
        in_specs=[
            pl.BlockSpec((rb, f), lambda i: (i, 0)),
            pl.BlockSpec((rb, 8), lambda i: (i, 0)),
            pl.BlockSpec((f, c), lambda i: (0, 0)),
            pl.BlockSpec((8, c), lambda i: (0, 0)),
            pl.BlockSpec((1, c), lambda i: (0, 0)),
        ],
        out_specs=pl.BlockSpec((rb, c), lambda i: (i, 0)),
        out_shape=jax.ShapeDtypeStruct((n, c), jnp.float32),
    )(x, agg, w1, w2e, b.reshape(1, c))


def _out2_kernel(y_ref, agg_ref, w2_ref, b_ref, o_ref):
    acc = y_ref[...] + (_dotb(agg_ref[...], w2_ref[...]) + b_ref[...])
    o_ref[...] = jnp.maximum(acc, 0.0)


def _out_layer2(y, agg, w2e, b, rb):
    n, c = y.shape
    return pl.pallas_call(
        _out2_kernel,
        grid=(n // rb,),
        in_specs=[
            pl.BlockSpec((rb, c), lambda i: (i, 0)),
            pl.BlockSpec((rb, 8), lambda i: (i, 0)),
            pl.BlockSpec((8, c), lambda i: (0, 0)),
            pl.BlockSpec((1, c), lambda i: (0, 0)),
        ],
        out_specs=pl.BlockSpec((rb, c), lambda i: (i, 0)),
        out_shape=jax.ShapeDtypeStruct((n, c), jnp.float32),
    )(y, agg, w2e, b.reshape(1, c))


def _make_scT(sh, n, npad):
    colsT = sh[:, 0:8].T if sh.shape[1] >= 8 else jnp.pad(sh, ((0, 0), (0, 8 - sh.shape[1]))).T
    pad = jnp.concatenate(
        [jnp.full((4, npad - n), 1e15, jnp.float32),
         jnp.zeros((4, npad - n), jnp.float32)], axis=0)
    return jnp.concatenate([colsT, pad], axis=1)


def _pad6to8(w2):
    return jnp.pad(w2, ((0, 2), (0, 0)))


def _knn_layer(sh, n, npad, rb_knn, sc_agg):
    scT = _make_scT(sh, n, npad)
    idx3 = _knn_idx3(sh, scT, rb_knn)
    idxT = jnp.concatenate(
        [idx3[:, 0:3].T, jnp.zeros((3, npad - n), jnp.int32)], axis=1)
    agg = sc_agg(scT.reshape(8 * npad), idxT.reshape(3 * npad))
    return agg[0:n]


def kernel(x, edge_index, Ws1, bs1, Wh1, bh1, Wo11, Wo21, bo21,
           Ws2, bs2, Wh2, bh2, Wo12, Wo22, bo22):
    del edge_index
    n = x.shape[0]
    npad = ((n + 511) // 512) * 512
    rb = _pick_rb(n, 1000)
    rb_knn = _pick_rb(n, 400)
    sc_agg = _make_sc_agg(npad)

    wsh1 = jnp.concatenate(
        [Ws1, Wh1, jnp.zeros((Ws1.shape[0], 1), jnp.float32)], axis=1)
    bsh1 = jnp.concatenate([bs1, bh1, jnp.zeros((1,), jnp.float32)])
    sh1 = _proj(x, wsh1, bsh1, _pick_rb(n, 1000))
    agg1 = _knn_layer(sh1, n, npad, rb_knn, sc_agg)

    hid = Wo11.shape[1]
    out_ch = Wo12.shape[1]
    w2cat = jnp.concatenate(
        [Ws2, Wh2, jnp.zeros((hid, 1), jnp.float32), Wo12], axis=1)
    b2cat = jnp.concatenate(
        [bs2, bh2, jnp.zeros((1 + out_ch,), jnp.float32)])
    shy2 = _out_proj_layer(x, agg1, Wo11, _pad6to8(Wo21), bo21,
                           w2cat, b2cat, rb)
    agg2 = _knn_layer(shy2, n, npad, rb_knn, sc_agg)
    y2a = shy2[:, 8:8 + out_ch]
    out = _out_layer2(y2a, agg2, _pad6to8(Wo22), bo22, _pick_rb(n, 1000))
    return out
